# Initial kernel scaffold; baseline (speedup 1.0000x reference)
import functools

import jax
import jax.numpy as jnp
from jax import lax
from jax.experimental import pallas as pl
from jax.experimental.pallas import tpu as pltpu
from jax.experimental.pallas import tpu_sc as plsc


def _sim_body(state_ref, rp_ref, mem_ref, max_ref, idx_ref, fit_ref, *,
              nb, heads, inv_b):
    i = pl.program_id(0)
    proj = lax.dot_general(
        state_ref[...], rp_ref[...], (((1,), (0,)), ((), ())),
        preferred_element_type=jnp.float32,
        precision=lax.Precision.HIGHEST)
    sims = lax.dot_general(
        mem_ref[...], proj, (((1,), (1,)), ((), ())),
        preferred_element_type=jnp.float32,
        precision=lax.Precision.HIGHEST)
    m = jnp.max(sims, axis=0, keepdims=True)
    row_iota = lax.broadcasted_iota(jnp.int32, sims.shape, 0)
    idx = jnp.min(jnp.where(sims == m, row_iota, heads), axis=0,
                  keepdims=True)
    max_ref[...] = m
    idx_ref[...] = idx

    @pl.when(i == 0)
    def _():
        fit_ref[...] = jnp.zeros_like(fit_ref)

    fit_ref[...] += jnp.sum(m, axis=1, keepdims=True)

    @pl.when(i == nb - 1)
    def _():
        fit_ref[...] = fit_ref[...] * inv_b


@functools.lru_cache(maxsize=None)
def _make_gather(v, d, b):
    info = plsc.get_sparse_core_info()
    nc, ns = info.num_cores, info.num_subcores
    nw = nc * ns
    assert b % (8 * nw) == 0 and d % info.num_lanes == 0
    b_per_w = b // nw
    mesh = plsc.VectorSubcoreMesh(core_axis_name="c", subcore_axis_name="s")

    @functools.partial(
        pl.kernel, mesh=mesh,
        out_type=jax.ShapeDtypeStruct((b, d), jnp.float32),
        scratch_types=[
            pltpu.VMEM((b_per_w,), jnp.int32),
            pltpu.VMEM((b_per_w, d), jnp.float32),
            pltpu.SemaphoreType.DMA,
        ],
    )
    def gather(table_hbm, idx_hbm, out_hbm, idx_v, rows_v, sem):
        wid = lax.axis_index("s") * nc + lax.axis_index("c")
        base = wid * b_per_w
        pltpu.sync_copy(idx_hbm.at[pl.ds(base, b_per_w)], idx_v)
        pltpu.async_copy(table_hbm.at[idx_v], rows_v, sem).wait()
        pltpu.sync_copy(rows_v, out_hbm.at[pl.ds(base, b_per_w)])

    return gather


def kernel(state, random_projection, memories, logits_table):
    b, in_dim = state.shape
    proj_dim = random_projection.shape[1]
    heads = memories.shape[0]
    bt = 256
    nb = b // bt

    maxs, idx, fit = pl.pallas_call(
        functools.partial(_sim_body, nb=nb, heads=heads, inv_b=1.0 / b),
        grid=(nb,),
        in_specs=[
            pl.BlockSpec((bt, in_dim), lambda i: (i, 0)),
            pl.BlockSpec((in_dim, proj_dim), lambda i: (0, 0)),
            pl.BlockSpec((heads, proj_dim), lambda i: (0, 0)),
        ],
        out_specs=[
            pl.BlockSpec((1, bt), lambda i: (0, i)),
            pl.BlockSpec((1, bt), lambda i: (0, i)),
            pl.BlockSpec((1, 1), lambda i: (0, 0)),
        ],
        out_shape=[
            jax.ShapeDtypeStruct((1, b), jnp.float32),
            jax.ShapeDtypeStruct((1, b), jnp.int32),
            jax.ShapeDtypeStruct((1, 1), jnp.float32),
        ],
    )(state, random_projection, memories)

    closest = idx.reshape(b)
    out_logits = _make_gather(heads, logits_table.shape[1], b)(
        logits_table, closest)
    return out_logits, fit[0, 0]


# speedup vs baseline: 1.2383x; 1.2383x over previous
import functools

import jax
import jax.numpy as jnp
from jax import lax
from jax.experimental import pallas as pl
from jax.experimental.pallas import tpu as pltpu
from jax.experimental.pallas import tpu_sc as plsc


def _sim_body(state_ref, rp_ref, mem_ref, max_ref, idx_ref, fit_ref, *,
              nb, heads, inv_b):
    i = pl.program_id(0)
    proj = lax.dot_general(
        state_ref[...], rp_ref[...], (((1,), (0,)), ((), ())),
        preferred_element_type=jnp.float32,
        precision=lax.Precision.DEFAULT)
    sims = lax.dot_general(
        mem_ref[...], proj, (((1,), (1,)), ((), ())),
        preferred_element_type=jnp.float32,
        precision=lax.Precision.DEFAULT)
    m = jnp.max(sims, axis=0, keepdims=True)
    row_iota = lax.broadcasted_iota(jnp.int32, sims.shape, 0)
    idx = jnp.min(jnp.where(sims == m, row_iota, heads), axis=0,
                  keepdims=True)
    max_ref[...] = m
    idx_ref[...] = idx

    @pl.when(i == 0)
    def _():
        fit_ref[...] = jnp.zeros_like(fit_ref)

    fit_ref[...] += jnp.sum(m, axis=1, keepdims=True)

    @pl.when(i == nb - 1)
    def _():
        fit_ref[...] = fit_ref[...] * inv_b


@functools.lru_cache(maxsize=None)
def _make_gather(v, d, b):
    info = plsc.get_sparse_core_info()
    nc, ns = info.num_cores, info.num_subcores
    nw = nc * ns
    assert b % (8 * nw) == 0 and d % info.num_lanes == 0
    b_per_w = b // nw
    mesh = plsc.VectorSubcoreMesh(core_axis_name="c", subcore_axis_name="s")

    @functools.partial(
        pl.kernel, mesh=mesh,
        out_type=jax.ShapeDtypeStruct((b, d), jnp.float32),
        scratch_types=[
            pltpu.VMEM((b_per_w,), jnp.int32),
            pltpu.VMEM((b_per_w, d), jnp.float32),
            pltpu.SemaphoreType.DMA,
        ],
    )
    def gather(table_hbm, idx_hbm, out_hbm, idx_v, rows_v, sem):
        wid = lax.axis_index("s") * nc + lax.axis_index("c")
        base = wid * b_per_w
        pltpu.sync_copy(idx_hbm.at[pl.ds(base, b_per_w)], idx_v)
        pltpu.async_copy(table_hbm.at[idx_v], rows_v, sem).wait()
        pltpu.sync_copy(rows_v, out_hbm.at[pl.ds(base, b_per_w)])

    return gather


def kernel(state, random_projection, memories, logits_table):
    b, in_dim = state.shape
    proj_dim = random_projection.shape[1]
    heads = memories.shape[0]
    bt = 256
    nb = b // bt

    maxs, idx, fit = pl.pallas_call(
        functools.partial(_sim_body, nb=nb, heads=heads, inv_b=1.0 / b),
        grid=(nb,),
        in_specs=[
            pl.BlockSpec((bt, in_dim), lambda i: (i, 0)),
            pl.BlockSpec((in_dim, proj_dim), lambda i: (0, 0)),
            pl.BlockSpec((heads, proj_dim), lambda i: (0, 0)),
        ],
        out_specs=[
            pl.BlockSpec((1, bt), lambda i: (0, i)),
            pl.BlockSpec((1, bt), lambda i: (0, i)),
            pl.BlockSpec((1, 1), lambda i: (0, 0)),
        ],
        out_shape=[
            jax.ShapeDtypeStruct((1, b), jnp.float32),
            jax.ShapeDtypeStruct((1, b), jnp.int32),
            jax.ShapeDtypeStruct((1, 1), jnp.float32),
        ],
    )(state, random_projection, memories)

    closest = idx.reshape(b)
    act_dim = logits_table.shape[1]
    pad = (-act_dim) % 128
    table = (jnp.pad(logits_table, ((0, 0), (0, pad))) if pad else
             logits_table)
    gathered = _make_gather(heads, table.shape[1], b)(table, closest)
    return gathered[:, :act_dim], fit[0, 0]


# speedup vs baseline: 1.6223x; 1.3101x over previous
import functools

import jax
import jax.numpy as jnp
from jax import lax
from jax.experimental import pallas as pl
from jax.experimental.pallas import tpu as pltpu
from jax.experimental.pallas import tpu_sc as plsc


def _sim_body(state_ref, rp_ref, mem_ref, max_ref, idx_ref, fit_ref, *,
              nb, heads, inv_b):
    i = pl.program_id(0)
    proj = lax.dot_general(
        state_ref[...], rp_ref[...], (((1,), (0,)), ((), ())),
        preferred_element_type=jnp.float32,
        precision=lax.Precision.DEFAULT)
    sims = lax.dot_general(
        mem_ref[...], proj, (((1,), (1,)), ((), ())),
        preferred_element_type=jnp.float32,
        precision=lax.Precision.DEFAULT)
    slots = 64
    bt = sims.shape[1]
    ngrp = heads // slots
    sims_r = sims.reshape(ngrp, slots, bt)
    vm = sims_r[0]
    vg = jnp.zeros((slots, bt), jnp.int32)
    for g in range(1, ngrp):
        c = sims_r[g]
        gt = c > vm
        vm = jnp.where(gt, c, vm)
        vg = jnp.where(gt, g, vg)
    vh = vg * slots + lax.broadcasted_iota(jnp.int32, (slots, bt), 0)
    m = jnp.max(vm, axis=0, keepdims=True)
    idx = jnp.min(jnp.where(vm == m, vh, heads), axis=0,
                  keepdims=True)
    max_ref[...] = m
    idx_ref[...] = idx

    @pl.when(i == 0)
    def _():
        fit_ref[...] = jnp.zeros_like(fit_ref)

    fit_ref[...] += jnp.sum(m, axis=1, keepdims=True)

    @pl.when(i == nb - 1)
    def _():
        fit_ref[...] = fit_ref[...] * inv_b


@functools.lru_cache(maxsize=None)
def _make_gather(v, d, b):
    info = plsc.get_sparse_core_info()
    nc, ns = info.num_cores, info.num_subcores
    nw = nc * ns
    assert b % (8 * nw) == 0 and d % info.num_lanes == 0
    b_per_w = b // nw
    mesh = plsc.VectorSubcoreMesh(core_axis_name="c", subcore_axis_name="s")

    @functools.partial(
        pl.kernel, mesh=mesh,
        out_type=jax.ShapeDtypeStruct((b, d), jnp.float32),
        scratch_types=[
            pltpu.VMEM((b_per_w,), jnp.int32),
            pltpu.VMEM((b_per_w, d), jnp.float32),
            pltpu.SemaphoreType.DMA,
        ],
    )
    def gather(table_hbm, idx_hbm, out_hbm, idx_v, rows_v, sem):
        wid = lax.axis_index("s") * nc + lax.axis_index("c")
        base = wid * b_per_w
        pltpu.sync_copy(idx_hbm.at[pl.ds(base, b_per_w)], idx_v)
        pltpu.async_copy(table_hbm.at[idx_v], rows_v, sem).wait()
        pltpu.sync_copy(rows_v, out_hbm.at[pl.ds(base, b_per_w)])

    return gather


def kernel(state, random_projection, memories, logits_table):
    b, in_dim = state.shape
    proj_dim = random_projection.shape[1]
    heads = memories.shape[0]
    bt = 256
    nb = b // bt

    maxs, idx, fit = pl.pallas_call(
        functools.partial(_sim_body, nb=nb, heads=heads, inv_b=1.0 / b),
        grid=(nb,),
        in_specs=[
            pl.BlockSpec((bt, in_dim), lambda i: (i, 0)),
            pl.BlockSpec((in_dim, proj_dim), lambda i: (0, 0)),
            pl.BlockSpec((heads, proj_dim), lambda i: (0, 0)),
        ],
        out_specs=[
            pl.BlockSpec((1, bt), lambda i: (0, i)),
            pl.BlockSpec((1, bt), lambda i: (0, i)),
            pl.BlockSpec((1, 1), lambda i: (0, 0)),
        ],
        out_shape=[
            jax.ShapeDtypeStruct((1, b), jnp.float32),
            jax.ShapeDtypeStruct((1, b), jnp.int32),
            jax.ShapeDtypeStruct((1, 1), jnp.float32),
        ],
    )(state, random_projection, memories)

    closest = idx.reshape(b)
    act_dim = logits_table.shape[1]
    pad = (-act_dim) % 128
    table = (jnp.pad(logits_table, ((0, 0), (0, pad))) if pad else
             logits_table)
    gathered = _make_gather(heads, table.shape[1], b)(table, closest)
    return gathered[:, :act_dim], fit[0, 0]


# speedup vs baseline: 1.9313x; 1.1904x over previous
import functools

import jax
import jax.numpy as jnp
from jax import lax
from jax.experimental import pallas as pl
from jax.experimental.pallas import tpu as pltpu
from jax.experimental.pallas import tpu_sc as plsc


def _sim_body(state_ref, rp_ref, mem_ref, max_ref, idx_ref, fit_ref, *,
              nb, heads, inv_b):
    i = pl.program_id(0)
    proj = lax.dot_general(
        state_ref[...], rp_ref[...], (((1,), (0,)), ((), ())),
        preferred_element_type=jnp.float32,
        precision=lax.Precision.DEFAULT)
    sims = lax.dot_general(
        mem_ref[...], proj, (((1,), (1,)), ((), ())),
        preferred_element_type=jnp.float32,
        precision=lax.Precision.DEFAULT)
    slots = 64
    bt = sims.shape[1]
    ngrp = heads // slots
    sims_r = sims.reshape(ngrp, slots, bt)
    vm = sims_r[0]
    vg = jnp.zeros((slots, bt), jnp.int32)
    for g in range(1, ngrp):
        c = sims_r[g]
        gt = c > vm
        vm = jnp.where(gt, c, vm)
        vg = jnp.where(gt, g, vg)
    vh = vg * slots + lax.broadcasted_iota(jnp.int32, (slots, bt), 0)
    m = jnp.max(vm, axis=0, keepdims=True)
    idx = jnp.min(jnp.where(vm == m, vh, heads), axis=0,
                  keepdims=True)
    max_ref[...] = m
    idx_ref[...] = idx

    @pl.when(i == 0)
    def _():
        fit_ref[...] = jnp.zeros_like(fit_ref)

    fit_ref[...] += jnp.sum(m, axis=1, keepdims=True)

    @pl.when(i == nb - 1)
    def _():
        fit_ref[...] = fit_ref[...] * inv_b


@functools.lru_cache(maxsize=None)
def _make_gather(v, d, b):
    info = plsc.get_sparse_core_info()
    nc, ns = info.num_cores, info.num_subcores
    nw = nc * ns
    assert b % (8 * nw) == 0 and d % info.num_lanes == 0
    b_per_w = b // nw
    mesh = plsc.VectorSubcoreMesh(core_axis_name="c", subcore_axis_name="s")

    @functools.partial(
        pl.kernel, mesh=mesh,
        out_type=jax.ShapeDtypeStruct((b, d), jnp.float32),
        scratch_types=[
            pltpu.VMEM((b_per_w,), jnp.int32),
            pltpu.VMEM((b_per_w, d), jnp.float32),
            pltpu.SemaphoreType.DMA,
        ],
    )
    def gather(table_hbm, idx_hbm, out_hbm, idx_v, rows_v, sem):
        wid = lax.axis_index("s") * nc + lax.axis_index("c")
        base = wid * b_per_w
        pltpu.sync_copy(idx_hbm.at[pl.ds(base, b_per_w)], idx_v)
        pltpu.async_copy(table_hbm.at[idx_v], rows_v, sem).wait()
        pltpu.sync_copy(rows_v, out_hbm.at[pl.ds(base, b_per_w)])

    return gather


def kernel(state, random_projection, memories, logits_table):
    b, in_dim = state.shape
    proj_dim = random_projection.shape[1]
    heads = memories.shape[0]
    bt = 512
    nb = b // bt

    maxs, idx, fit = pl.pallas_call(
        functools.partial(_sim_body, nb=nb, heads=heads, inv_b=1.0 / b),
        grid=(nb,),
        in_specs=[
            pl.BlockSpec((bt, in_dim), lambda i: (i, 0)),
            pl.BlockSpec((in_dim, proj_dim), lambda i: (0, 0)),
            pl.BlockSpec((heads, proj_dim), lambda i: (0, 0)),
        ],
        out_specs=[
            pl.BlockSpec((1, bt), lambda i: (0, i)),
            pl.BlockSpec((1, bt), lambda i: (0, i)),
            pl.BlockSpec((1, 1), lambda i: (0, 0)),
        ],
        out_shape=[
            jax.ShapeDtypeStruct((1, b), jnp.float32),
            jax.ShapeDtypeStruct((1, b), jnp.int32),
            jax.ShapeDtypeStruct((1, 1), jnp.float32),
        ],
    )(state, random_projection, memories)

    closest = idx.reshape(b)
    act_dim = logits_table.shape[1]
    pad = (-act_dim) % 128
    table = (jnp.pad(logits_table, ((0, 0), (0, pad))) if pad else
             logits_table)
    gathered = _make_gather(heads, table.shape[1], b)(table, closest)
    return gathered[:, :act_dim], fit[0, 0]


# speedup vs baseline: 2.0073x; 1.0394x over previous
import functools

import jax
import jax.numpy as jnp
from jax import lax
from jax.experimental import pallas as pl
from jax.experimental.pallas import tpu as pltpu
from jax.experimental.pallas import tpu_sc as plsc


def _sim_body(state_ref, rp_ref, mem_ref, max_ref, idx_ref, fit_ref, *,
              nb, heads, inv_b):
    i = pl.program_id(0)
    proj = lax.dot_general(
        state_ref[...], rp_ref[...], (((1,), (0,)), ((), ())),
        preferred_element_type=jnp.float32,
        precision=lax.Precision.DEFAULT)
    sims = lax.dot_general(
        mem_ref[...], proj, (((1,), (1,)), ((), ())),
        preferred_element_type=jnp.float32,
        precision=lax.Precision.DEFAULT)
    slots = 64
    bt = sims.shape[1]
    ngrp = heads // slots
    sims_r = sims.reshape(ngrp, slots, bt)
    vm = sims_r[0]
    vg = jnp.zeros((slots, bt), jnp.int32)
    for g in range(1, ngrp):
        c = sims_r[g]
        gt = c > vm
        vm = jnp.where(gt, c, vm)
        vg = jnp.where(gt, g, vg)
    vh = vg * slots + lax.broadcasted_iota(jnp.int32, (slots, bt), 0)
    m = jnp.max(vm, axis=0, keepdims=True)
    idx = jnp.min(jnp.where(vm == m, vh, heads), axis=0,
                  keepdims=True)
    max_ref[...] = m
    idx_ref[...] = idx

    @pl.when(i == 0)
    def _():
        fit_ref[...] = jnp.zeros_like(fit_ref)

    fit_ref[...] += jnp.sum(m, axis=1, keepdims=True)

    @pl.when(i == nb - 1)
    def _():
        fit_ref[...] = fit_ref[...] * inv_b


@functools.lru_cache(maxsize=None)
def _make_gather(v, d, b):
    info = plsc.get_sparse_core_info()
    nc, ns = info.num_cores, info.num_subcores
    nw = nc * ns
    assert b % (8 * nw) == 0 and d % info.num_lanes == 0
    b_per_w = b // nw
    mesh = plsc.VectorSubcoreMesh(core_axis_name="c", subcore_axis_name="s")

    @functools.partial(
        pl.kernel, mesh=mesh,
        out_type=jax.ShapeDtypeStruct((b, d), jnp.float32),
        scratch_types=[
            pltpu.VMEM((b_per_w,), jnp.int32),
            pltpu.VMEM((b_per_w, d), jnp.float32),
            pltpu.SemaphoreType.DMA,
        ],
    )
    def gather(table_hbm, idx_hbm, out_hbm, idx_v, rows_v, sem):
        wid = lax.axis_index("s") * nc + lax.axis_index("c")
        base = wid * b_per_w
        pltpu.sync_copy(idx_hbm.at[pl.ds(base, b_per_w)], idx_v)
        pltpu.async_copy(table_hbm.at[idx_v], rows_v, sem).wait()
        pltpu.sync_copy(rows_v, out_hbm.at[pl.ds(base, b_per_w)])

    return gather


def kernel(state, random_projection, memories, logits_table):
    b, in_dim = state.shape
    proj_dim = random_projection.shape[1]
    heads = memories.shape[0]
    bt = 1024
    nb = b // bt

    maxs, idx, fit = pl.pallas_call(
        functools.partial(_sim_body, nb=nb, heads=heads, inv_b=1.0 / b),
        grid=(nb,),
        in_specs=[
            pl.BlockSpec((bt, in_dim), lambda i: (i, 0)),
            pl.BlockSpec((in_dim, proj_dim), lambda i: (0, 0)),
            pl.BlockSpec((heads, proj_dim), lambda i: (0, 0)),
        ],
        out_specs=[
            pl.BlockSpec((1, bt), lambda i: (0, i)),
            pl.BlockSpec((1, bt), lambda i: (0, i)),
            pl.BlockSpec((1, 1), lambda i: (0, 0)),
        ],
        out_shape=[
            jax.ShapeDtypeStruct((1, b), jnp.float32),
            jax.ShapeDtypeStruct((1, b), jnp.int32),
            jax.ShapeDtypeStruct((1, 1), jnp.float32),
        ],
    )(state, random_projection, memories)

    closest = idx.reshape(b)
    act_dim = logits_table.shape[1]
    pad = (-act_dim) % 128
    table = (jnp.pad(logits_table, ((0, 0), (0, pad))) if pad else
             logits_table)
    gathered = _make_gather(heads, table.shape[1], b)(table, closest)
    return gathered[:, :act_dim], fit[0, 0]


# speedup vs baseline: 2.0464x; 1.0195x over previous
import functools

import jax
import jax.numpy as jnp
from jax import lax
from jax.experimental import pallas as pl
from jax.experimental.pallas import tpu as pltpu
from jax.experimental.pallas import tpu_sc as plsc


def _sim_body(state_ref, rp_ref, mem_ref, max_ref, idx_ref, fit_ref, *,
              nb, heads, inv_b):
    i = pl.program_id(0)
    proj = lax.dot_general(
        state_ref[...], rp_ref[...], (((1,), (0,)), ((), ())),
        preferred_element_type=jnp.float32,
        precision=lax.Precision.DEFAULT)
    sims = lax.dot_general(
        mem_ref[...], proj, (((1,), (1,)), ((), ())),
        preferred_element_type=jnp.float32,
        precision=lax.Precision.DEFAULT)
    slots = 64
    bt = sims.shape[1]
    ngrp = heads // slots
    sims_r = sims.reshape(ngrp, slots, bt)
    vm = sims_r[0]
    vg = jnp.zeros((slots, bt), jnp.int32)
    for g in range(1, ngrp):
        c = sims_r[g]
        gt = c > vm
        vm = jnp.where(gt, c, vm)
        vg = jnp.where(gt, g, vg)
    vh = vg * slots + lax.broadcasted_iota(jnp.int32, (slots, bt), 0)
    m = jnp.max(vm, axis=0, keepdims=True)
    idx = jnp.min(jnp.where(vm == m, vh, heads), axis=0,
                  keepdims=True)
    max_ref[...] = m
    idx_ref[...] = idx

    @pl.when(i == 0)
    def _():
        fit_ref[...] = jnp.zeros_like(fit_ref)

    fit_ref[...] += jnp.sum(m, axis=1, keepdims=True)

    @pl.when(i == nb - 1)
    def _():
        fit_ref[...] = fit_ref[...] * inv_b


@functools.lru_cache(maxsize=None)
def _make_gather(v, d, b):
    info = plsc.get_sparse_core_info()
    nc, ns = info.num_cores, info.num_subcores
    nw = nc * ns
    assert b % (8 * nw) == 0 and d % info.num_lanes == 0
    b_per_w = b // nw
    mesh = plsc.VectorSubcoreMesh(core_axis_name="c", subcore_axis_name="s")

    @functools.partial(
        pl.kernel, mesh=mesh,
        compiler_params=pltpu.CompilerParams(use_tc_tiling_on_sc=False),
        out_type=jax.ShapeDtypeStruct((b, d), jnp.float32),
        scratch_types=[
            pltpu.VMEM((b_per_w,), jnp.int32),
            pltpu.VMEM((b_per_w, d), jnp.float32),
            pltpu.SemaphoreType.DMA,
        ],
    )
    def gather(table_hbm, idx_hbm, out_hbm, idx_v, rows_v, sem):
        wid = lax.axis_index("s") * nc + lax.axis_index("c")
        base = wid * b_per_w
        pltpu.sync_copy(idx_hbm.at[pl.ds(base, b_per_w)], idx_v)
        pltpu.async_copy(table_hbm.at[idx_v], rows_v, sem).wait()
        pltpu.sync_copy(rows_v, out_hbm.at[pl.ds(base, b_per_w)])

    return gather


def kernel(state, random_projection, memories, logits_table):
    b, in_dim = state.shape
    proj_dim = random_projection.shape[1]
    heads = memories.shape[0]
    bt = 1024
    nb = b // bt

    maxs, idx, fit = pl.pallas_call(
        functools.partial(_sim_body, nb=nb, heads=heads, inv_b=1.0 / b),
        grid=(nb,),
        in_specs=[
            pl.BlockSpec((bt, in_dim), lambda i: (i, 0)),
            pl.BlockSpec((in_dim, proj_dim), lambda i: (0, 0)),
            pl.BlockSpec((heads, proj_dim), lambda i: (0, 0)),
        ],
        out_specs=[
            pl.BlockSpec((1, bt), lambda i: (0, i)),
            pl.BlockSpec((1, bt), lambda i: (0, i)),
            pl.BlockSpec((1, 1), lambda i: (0, 0)),
        ],
        out_shape=[
            jax.ShapeDtypeStruct((1, b), jnp.float32),
            jax.ShapeDtypeStruct((1, b), jnp.int32),
            jax.ShapeDtypeStruct((1, 1), jnp.float32),
        ],
    )(state, random_projection, memories)

    closest = idx.reshape(b)
    out_logits = _make_gather(heads, logits_table.shape[1], b)(
        logits_table, closest)
    return out_logits, fit[0, 0]
